# baseline (speedup 1.0000x reference)
import jax
import jax.numpy as jnp
from jax.experimental import pallas as pl
from jax.experimental.pallas import tpu as pltpu

T = 128
D = 1024
E = 64
DFF = 512
NBUF = 5


def _nt_dot(a, b):
    return jax.lax.dot_general(
        a, b, (((1,), (1,)), ((), ())), preferred_element_type=jnp.float32
    )


def _copy_in(w1_hbm, w2_hbm, w3_hbm, b1, b2, b3, sems, e, slot):
    pltpu.make_async_copy(w1_hbm.at[e], b1.at[slot], sems.at[slot, 0]).start()
    pltpu.make_async_copy(w2_hbm.at[e], b2.at[slot], sems.at[slot, 1]).start()
    pltpu.make_async_copy(w3_hbm.at[e], b3.at[slot], sems.at[slot, 2]).start()


def _moe_kernel(x_ref, gw_ref, w1_hbm, w2_hbm, w3_hbm,
                sw1_ref, sw2_ref, sw3_ref,
                out_ref, gates_ref, acc_ref,
                b1, b2, b3, sems):
    x = x_ref[...]

    logits = _nt_dot(x, gw_ref[...])
    m = jnp.max(logits, axis=-1, keepdims=True)
    ex = jnp.exp(logits - m)
    probs = ex / jnp.sum(ex, axis=-1, keepdims=True)
    col = jax.lax.broadcasted_iota(jnp.int32, (T, E), 1)
    m1 = jnp.max(probs, axis=-1, keepdims=True)
    idx1 = jnp.min(jnp.where(probs == m1, col, E), axis=-1, keepdims=True)
    oh1 = col == idx1
    probs_m = jnp.where(oh1, -1.0, probs)
    m2 = jnp.max(probs_m, axis=-1, keepdims=True)
    idx2 = jnp.min(jnp.where(probs_m == m2, col, E), axis=-1, keepdims=True)
    oh2 = col == idx2
    gates = (jnp.where(oh1, m1, 0.0) + jnp.where(oh2, m2, 0.0)) / (m1 + m2)
    gates_ref[...] = gates

    for s in range(NBUF):
        @pl.when(jnp.sum(jnp.where(col == s, gates, 0.0)) > 0.0)
        def _():
            _copy_in(w1_hbm, w2_hbm, w3_hbm, b1, b2, b3, sems, s, s)

    sh1 = _nt_dot(x, sw1_ref[...])
    sh3 = _nt_dot(x, sw3_ref[...])
    sh = (sh1 * jax.nn.sigmoid(sh1)) * sh3
    acc_ref[...] = _nt_dot(sh, sw2_ref[...])

    def body(e, carry):
        slot = jax.lax.rem(e, NBUF)
        g = jnp.sum(jnp.where(col == e, gates_ref[...], 0.0),
                    axis=1, keepdims=True)

        @pl.when(jnp.sum(g) > 0.0)
        def _():
            pltpu.make_async_copy(w1_hbm.at[e], b1.at[slot], sems.at[slot, 0]).wait()
            pltpu.make_async_copy(w2_hbm.at[e], b2.at[slot], sems.at[slot, 1]).wait()
            pltpu.make_async_copy(w3_hbm.at[e], b3.at[slot], sems.at[slot, 2]).wait()
            w1 = b1[slot]
            w2 = b2[slot]
            w3 = b3[slot]
            h1 = _nt_dot(x, w1)
            h3 = _nt_dot(x, w3)
            h = (h1 * jax.nn.sigmoid(h1)) * h3
            acc_ref[...] += _nt_dot(h * g, w2)

        nxt = e + NBUF
        gn = jnp.sum(jnp.where(col == nxt, gates_ref[...], 0.0))
        @pl.when((nxt < E) & (gn > 0.0))
        def _():
            _copy_in(w1_hbm, w2_hbm, w3_hbm, b1, b2, b3, sems, nxt, slot)
        return carry

    jax.lax.fori_loop(0, E, body, 0, unroll=2)
    out_ref[...] = acc_ref[...]


def kernel(hidden_states, gate_w, w1, w2, w3, s_w1, s_w2, s_w3):
    vmem = lambda: pl.BlockSpec(memory_space=pltpu.MemorySpace.VMEM)
    anym = lambda: pl.BlockSpec(memory_space=pl.ANY)
    return pl.pallas_call(
        _moe_kernel,
        in_specs=[
            vmem(),
            vmem(),
            anym(),
            anym(),
            anym(),
            vmem(),
            vmem(),
            vmem(),
        ],
        out_specs=vmem(),
        out_shape=jax.ShapeDtypeStruct((T, D), jnp.float32),
        scratch_shapes=[
            pltpu.VMEM((T, E), jnp.float32),
            pltpu.VMEM((T, D), jnp.float32),
            pltpu.VMEM((NBUF, DFF, D), jnp.float32),
            pltpu.VMEM((NBUF, D, DFF), jnp.float32),
            pltpu.VMEM((NBUF, DFF, D), jnp.float32),
            pltpu.SemaphoreType.DMA((NBUF, 3)),
        ],
    )(hidden_states, gate_w, w1, w2, w3, s_w1, s_w2, s_w3)


# speedup vs baseline: 1.0214x; 1.0214x over previous
import jax
import jax.numpy as jnp
from jax.experimental import pallas as pl
from jax.experimental.pallas import tpu as pltpu

T = 128
D = 1024
E = 64
DFF = 512
NBUF = 4


def _nt_dot(a, b):
    return jax.lax.dot_general(
        a, b, (((1,), (1,)), ((), ())), preferred_element_type=jnp.float32
    )


def _copy_in(w1_hbm, w2_hbm, w3_hbm, b1, b2, b3, sems, e, slot):
    pltpu.make_async_copy(w1_hbm.at[e], b1.at[slot], sems.at[slot, 0]).start()
    pltpu.make_async_copy(w2_hbm.at[e], b2.at[slot], sems.at[slot, 1]).start()
    pltpu.make_async_copy(w3_hbm.at[e], b3.at[slot], sems.at[slot, 2]).start()


def _moe_kernel(x_ref, gw_ref, w1_hbm, w2_hbm, w3_hbm,
                sw1_ref, sw2_ref, sw3_ref,
                out_ref, gates_ref, acc_ref,
                b1, b2, b3, sems):
    x = x_ref[...]

    logits = _nt_dot(x, gw_ref[...])
    m = jnp.max(logits, axis=-1, keepdims=True)
    ex = jnp.exp(logits - m)
    probs = ex / jnp.sum(ex, axis=-1, keepdims=True)
    col = jax.lax.broadcasted_iota(jnp.int32, (T, E), 1)
    m1 = jnp.max(probs, axis=-1, keepdims=True)
    idx1 = jnp.min(jnp.where(probs == m1, col, E), axis=-1, keepdims=True)
    oh1 = col == idx1
    probs_m = jnp.where(oh1, -1.0, probs)
    m2 = jnp.max(probs_m, axis=-1, keepdims=True)
    idx2 = jnp.min(jnp.where(probs_m == m2, col, E), axis=-1, keepdims=True)
    oh2 = col == idx2
    gates = (jnp.where(oh1, m1, 0.0) + jnp.where(oh2, m2, 0.0)) / (m1 + m2)
    gates_ref[...] = gates

    for s in range(NBUF):
        @pl.when(jnp.sum(jnp.where(col == s, gates, 0.0)) > 0.0)
        def _():
            _copy_in(w1_hbm, w2_hbm, w3_hbm, b1, b2, b3, sems, s, s)

    sh1 = _nt_dot(x, sw1_ref[...])
    sh3 = _nt_dot(x, sw3_ref[...])
    sh = (sh1 * jax.nn.sigmoid(sh1)) * sh3
    acc_ref[...] = _nt_dot(sh, sw2_ref[...])

    def body(e, carry):
        slot = jax.lax.rem(e, NBUF)
        g = jnp.sum(jnp.where(col == e, gates_ref[...], 0.0),
                    axis=1, keepdims=True)

        @pl.when(jnp.sum(g) > 0.0)
        def _():
            pltpu.make_async_copy(w1_hbm.at[e], b1.at[slot], sems.at[slot, 0]).wait()
            pltpu.make_async_copy(w2_hbm.at[e], b2.at[slot], sems.at[slot, 1]).wait()
            pltpu.make_async_copy(w3_hbm.at[e], b3.at[slot], sems.at[slot, 2]).wait()
            w1 = b1[slot]
            w2 = b2[slot]
            w3 = b3[slot]
            h1 = _nt_dot(x, w1)
            h3 = _nt_dot(x, w3)
            h = (h1 * jax.nn.sigmoid(h1)) * h3
            acc_ref[...] += _nt_dot(h * g, w2)

        nxt = e + NBUF
        gn = jnp.sum(jnp.where(col == nxt, gates_ref[...], 0.0))
        @pl.when((nxt < E) & (gn > 0.0))
        def _():
            _copy_in(w1_hbm, w2_hbm, w3_hbm, b1, b2, b3, sems, nxt, slot)
        return carry

    jax.lax.fori_loop(0, E, body, 0, unroll=2)
    out_ref[...] = acc_ref[...]


def kernel(hidden_states, gate_w, w1, w2, w3, s_w1, s_w2, s_w3):
    vmem = lambda: pl.BlockSpec(memory_space=pltpu.MemorySpace.VMEM)
    anym = lambda: pl.BlockSpec(memory_space=pl.ANY)
    return pl.pallas_call(
        _moe_kernel,
        in_specs=[
            vmem(),
            vmem(),
            anym(),
            anym(),
            anym(),
            vmem(),
            vmem(),
            vmem(),
        ],
        out_specs=vmem(),
        out_shape=jax.ShapeDtypeStruct((T, D), jnp.float32),
        scratch_shapes=[
            pltpu.VMEM((T, E), jnp.float32),
            pltpu.VMEM((T, D), jnp.float32),
            pltpu.VMEM((NBUF, DFF, D), jnp.float32),
            pltpu.VMEM((NBUF, D, DFF), jnp.float32),
            pltpu.VMEM((NBUF, DFF, D), jnp.float32),
            pltpu.SemaphoreType.DMA((NBUF, 3)),
        ],
    )(hidden_states, gate_w, w1, w2, w3, s_w1, s_w2, s_w3)
